# baseline (speedup 1.0000x reference)
import jax
import jax.numpy as jnp
from jax.experimental import pallas as pl

_DIM = 256
_K = 1024
_NQ = 8
_CW = 0.25
_ROWS = 1024


def _dist_kernel(r_ref, cb_ref, d_ref):
    residual = r_ref[0]
    cb = cb_ref[0]
    scores = jax.lax.dot_general(
        residual, cb, (((1,), (1,)), ((), ())),
        preferred_element_type=jnp.float32)
    a2 = jnp.sum(residual * residual, axis=1, keepdims=True)
    b2 = jnp.sum(cb * cb, axis=1)[None, :]
    val = (a2 + b2) - 2.0 * scores
    d_ref[0] = jnp.where(val > 0.0, val * jax.lax.rsqrt(val), 0.0)


def _dists(residual_flat, cb_q):
    n = residual_flat.shape[0]
    g = n // _ROWS
    out = pl.pallas_call(
        _dist_kernel,
        grid=(g,),
        in_specs=[
            pl.BlockSpec((1, _ROWS, _DIM), lambda b: (b, 0, 0)),
            pl.BlockSpec((1, _K, _DIM), lambda b: (0, 0, 0)),
        ],
        out_specs=pl.BlockSpec((1, _ROWS, _K), lambda b: (b, 0, 0)),
        out_shape=jax.ShapeDtypeStruct((g, _ROWS, _K), jnp.float32),
    )(residual_flat.reshape(g, _ROWS, _DIM), cb_q[None])
    return out.reshape(n, _K)


def kernel(z, codebooks):
    B, D, T = z.shape
    z_btd = jnp.transpose(z, (0, 2, 1))
    quantized = jnp.zeros_like(z_btd)
    residual = z_btd
    codes = []
    total_loss = jnp.asarray(0.0, dtype=jnp.float32)
    for q in range(_NQ):
        distances = _dists(residual.reshape(B * T, D), codebooks[q])
        indices = jnp.argmin(distances.reshape(B, T, _K), axis=-1)
        codes.append(indices)
        quantized_step = jnp.take(codebooks[q], indices, axis=0)
        quantized = quantized + quantized_step
        commitment_loss = jnp.mean((residual - quantized_step) ** 2)
        commitment_loss = jnp.clip(commitment_loss, 0.0, 10.0)
        total_loss = total_loss + commitment_loss * _CW
        residual = residual - quantized_step
    quantized_bdt = jnp.transpose(quantized, (0, 2, 1))
    codes_arr = jnp.stack(codes, axis=1)
    total_loss = jnp.clip(total_loss, 0.0, 10.0)
    return quantized_bdt, codes_arr, total_loss
